# baseline (speedup 1.0000x reference)
import functools

import jax
import jax.numpy as jnp
from jax import lax
from jax.experimental import pallas as pl
from jax.experimental.pallas import tpu as pltpu
from jax.experimental.pallas import tpu_sc as plsc

EPS = 1e-12
LANES = 16


def _rsqrt_s(x):
    i = lax.bitcast_convert_type(x, jnp.int32)
    i = jnp.int32(0x5F3759DF) - lax.shift_right_logical(i, 1)
    r = lax.bitcast_convert_type(i, jnp.float32)
    h = 0.5 * x
    for _ in range(1):
        r = r * (1.5 - h * r * r)
    return r


def kernel(input_ids, word_table, pos_table, tok_table, gamma, beta):
    B, L = input_ids.shape
    V, H = word_table.shape
    NW = 32
    HALF = L // 2
    RPB = B // NW
    NK = H // LANES

    ids = input_ids.astype(jnp.int32).reshape(B, 2, HALF)
    mesh = plsc.VectorSubcoreMesh(core_axis_name="c", subcore_axis_name="s")

    @functools.partial(
        pl.kernel,
        out_type=jax.ShapeDtypeStruct((B, L, H), jnp.float32),
        mesh=mesh,
        compiler_params=pltpu.CompilerParams(needs_layout_passes=False),
        scratch_types=[
            pltpu.VMEM((RPB, 2, HALF), jnp.int32),
            pltpu.VMEM((HALF, H), jnp.float32),
            pltpu.VMEM((HALF, H), jnp.float32),
            pltpu.VMEM((L, H), jnp.float32),
            pltpu.VMEM((L, H), jnp.float32),
            pltpu.VMEM((L, H), jnp.float32),
            pltpu.VMEM((H,), jnp.float32),
            pltpu.VMEM((H,), jnp.float32),
            pltpu.VMEM((H,), jnp.float32),
            pltpu.SemaphoreType.DMA,
            pltpu.SemaphoreType.DMA,
            pltpu.SemaphoreType.DMA,
            pltpu.SemaphoreType.DMA,
        ],
    )
    def sc_fn(ids_h, wt_h, pos_h, tok_h, g_h, b_h, out_h,
              ids_v, buf0_v, buf1_v, obuf0_v, obuf1_v,
              bias_v, tok_v, g_v, b_v, sin0, sin1, sout0, sout1):
        cid = lax.axis_index("c")
        sid = lax.axis_index("s")
        wid = sid * 2 + cid
        rbase = wid * RPB

        bufs = ((buf0_v, sin0), (buf1_v, sin1))
        obufs = ((obuf0_v, sout0), (obuf1_v, sout1))

        pltpu.sync_copy(ids_h.at[pl.ds(rbase, RPB)], ids_v)
        pltpu.sync_copy(g_h, g_v)
        pltpu.sync_copy(b_h, b_v)
        pltpu.sync_copy(tok_h.at[0], tok_v)
        pltpu.sync_copy(pos_h.at[pl.ds(0, L)], bias_v)

        @plsc.parallel_loop(0, L)
        def _(t):
            for k in range(NK):
                s = pl.ds(k * LANES, LANES)
                bias_v[t, s] = bias_v[t, s] + tok_v[s]

        def start_gather(r, b, buf_v, sin):
            pltpu.async_copy(wt_h.at[ids_v.at[r, b]], buf_v, sin)

        def wait_gather(r, b, buf_v, sin):
            pltpu.make_async_copy(wt_h.at[ids_v.at[r, b]], buf_v, sin).wait()

        def compute_half(buf_v, obuf_v, b):
            boff = b * HALF
            gb = tuple(g_v[pl.ds(k * LANES, LANES)] for k in range(NK)) \
                + tuple(b_v[pl.ds(k * LANES, LANES)] for k in range(NK))

            @plsc.parallel_loop(0, HALF, carry=gb)
            def _(j, gb_c):
                ys = []
                for k in range(NK):
                    s = pl.ds(k * LANES, LANES)
                    ys.append(buf_v[j, s] + bias_v[boff + j, s])
                t4 = (((ys[0] + ys[1]) + (ys[2] + ys[3]))
                      + ((ys[4] + ys[5]) + (ys[6] + ys[7])))
                ssum = plsc.cumsum(t4)[LANES - 1]
                sqs = [y * y for y in ys]
                q4 = (((sqs[0] + sqs[1]) + (sqs[2] + sqs[3]))
                      + ((sqs[4] + sqs[5]) + (sqs[6] + sqs[7])))
                ssq = plsc.cumsum(q4)[LANES - 1]
                mean = ssum * (1.0 / H)
                var = ssq * (1.0 / H) - mean * mean
                inv = jnp.full((LANES,), _rsqrt_s(var + EPS), dtype=jnp.float32)
                for k in range(NK):
                    s = pl.ds(k * LANES, LANES)
                    obuf_v[boff + j, s] = ((ys[k] - mean) * (inv * gb_c[k])
                                           + gb_c[NK + k])
                return gb_c

        start_gather(0, 0, buf0_v, sin0)
        start_gather(0, 1, buf1_v, sin1)

        def pair_body(q, carry):
            for i in range(2):
                obuf_v, sout = obufs[i]
                r = 2 * q + i
                row = rbase + r
                for b in range(2):
                    buf_v, sin = bufs[b]
                    wait_gather(r, b, buf_v, sin)
                    if b == 0:
                        @pl.when(q > 0)
                        def _():
                            pltpu.make_async_copy(
                                obuf_v, out_h.at[row], sout).wait()
                    compute_half(buf_v, obuf_v, b)
                    if i == 0:
                        start_gather(r + 1, b, buf_v, sin)
                    else:
                        @pl.when(q < RPB // 2 - 1)
                        def _():
                            start_gather(r + 1, b, buf_v, sin)
                pltpu.async_copy(obuf_v, out_h.at[row], sout)
            return carry

        lax.fori_loop(0, RPB // 2, pair_body, 0)

        pltpu.make_async_copy(obuf0_v, out_h.at[rbase + RPB - 2], sout0).wait()
        pltpu.make_async_copy(obuf1_v, out_h.at[rbase + RPB - 1], sout1).wait()

    return sc_fn(ids, word_table, pos_table, tok_table, gamma, beta)


# speedup vs baseline: 1.1070x; 1.1070x over previous
import functools

import jax
import jax.numpy as jnp
from jax import lax
from jax.experimental import pallas as pl
from jax.experimental.pallas import tpu as pltpu
from jax.experimental.pallas import tpu_sc as plsc

EPS = 1e-12
LANES = 16


def _rsqrt16(x):
    v = jnp.full((LANES,), x, dtype=jnp.float32)
    i = plsc.bitcast(v, jnp.int32)
    i = jnp.int32(0x5F3759DF) - lax.shift_right_logical(i, 1)
    r = plsc.bitcast(i, jnp.float32)
    for _ in range(1):
        r = r * (1.5 - 0.5 * v * r * r)
    return r


def kernel(input_ids, word_table, pos_table, tok_table, gamma, beta):
    B, L = input_ids.shape
    V, H = word_table.shape
    NW = 32
    HALF = L // 2
    RPB = B // NW
    NK = H // LANES

    ids = input_ids.astype(jnp.int32).reshape(B, 2, HALF)
    mesh = plsc.VectorSubcoreMesh(core_axis_name="c", subcore_axis_name="s")

    @functools.partial(
        pl.kernel,
        out_type=jax.ShapeDtypeStruct((B, L, H), jnp.float32),
        mesh=mesh,
        compiler_params=pltpu.CompilerParams(needs_layout_passes=False),
        scratch_types=[
            pltpu.VMEM((RPB, 2, HALF), jnp.int32),
            pltpu.VMEM((HALF, H), jnp.float32),
            pltpu.VMEM((HALF, H), jnp.float32),
            pltpu.VMEM((L, H), jnp.float32),
            pltpu.VMEM((L, H), jnp.float32),
            pltpu.VMEM((L, H), jnp.float32),
            pltpu.VMEM((H,), jnp.float32),
            pltpu.VMEM((H,), jnp.float32),
            pltpu.VMEM((H,), jnp.float32),
            pltpu.SemaphoreType.DMA,
            pltpu.SemaphoreType.DMA,
            pltpu.SemaphoreType.DMA,
            pltpu.SemaphoreType.DMA,
        ],
    )
    def sc_fn(ids_h, wt_h, pos_h, tok_h, g_h, b_h, out_h,
              ids_v, buf0_v, buf1_v, obuf0_v, obuf1_v,
              bias_v, tok_v, g_v, b_v, sin0, sin1, sout0, sout1):
        cid = lax.axis_index("c")
        sid = lax.axis_index("s")
        wid = sid * 2 + cid
        rbase = wid * RPB

        bufs = ((buf0_v, sin0), (buf1_v, sin1))
        obufs = ((obuf0_v, sout0), (obuf1_v, sout1))

        pltpu.sync_copy(ids_h.at[pl.ds(rbase, RPB)], ids_v)
        pltpu.sync_copy(g_h, g_v)
        pltpu.sync_copy(b_h, b_v)
        pltpu.sync_copy(tok_h.at[0], tok_v)
        pltpu.sync_copy(pos_h.at[pl.ds(0, L)], bias_v)

        @plsc.parallel_loop(0, L)
        def _(t):
            for k in range(NK):
                s = pl.ds(k * LANES, LANES)
                bias_v[t, s] = bias_v[t, s] + tok_v[s]

        def start_gather(r, b, buf_v, sin):
            pltpu.async_copy(wt_h.at[ids_v.at[r, b]], buf_v, sin)

        def wait_gather(r, b, buf_v, sin):
            pltpu.make_async_copy(wt_h.at[ids_v.at[r, b]], buf_v, sin).wait()

        def compute_half(buf_v, obuf_v, b):
            boff = b * HALF
            gb = tuple(g_v[pl.ds(k * LANES, LANES)] for k in range(NK)) \
                + tuple(b_v[pl.ds(k * LANES, LANES)] for k in range(NK))

            @plsc.parallel_loop(0, HALF, carry=gb)
            def _(j, gb_c):
                ys = []
                for k in range(NK):
                    s = pl.ds(k * LANES, LANES)
                    ys.append(buf_v[j, s] + bias_v[boff + j, s])
                t4 = (((ys[0] + ys[1]) + (ys[2] + ys[3]))
                      + ((ys[4] + ys[5]) + (ys[6] + ys[7])))
                ssum = plsc.cumsum(t4)[LANES - 1]
                sqs = [y * y for y in ys]
                q4 = (((sqs[0] + sqs[1]) + (sqs[2] + sqs[3]))
                      + ((sqs[4] + sqs[5]) + (sqs[6] + sqs[7])))
                ssq = plsc.cumsum(q4)[LANES - 1]
                mean = ssum * (1.0 / H)
                var = ssq * (1.0 / H) - mean * mean
                inv = _rsqrt16(var + EPS)
                for k in range(NK):
                    s = pl.ds(k * LANES, LANES)
                    obuf_v[boff + j, s] = ((ys[k] - mean) * (inv * gb_c[k])
                                           + gb_c[NK + k])
                return gb_c

        start_gather(0, 0, buf0_v, sin0)
        start_gather(0, 1, buf1_v, sin1)

        def pair_body(q, carry):
            for i in range(2):
                obuf_v, sout = obufs[i]
                r = 2 * q + i
                row = rbase + r
                for b in range(2):
                    buf_v, sin = bufs[b]
                    wait_gather(r, b, buf_v, sin)
                    if b == 0:
                        @pl.when(q > 0)
                        def _():
                            pltpu.make_async_copy(
                                obuf_v, out_h.at[row], sout).wait()
                    compute_half(buf_v, obuf_v, b)
                    if i == 0:
                        start_gather(r + 1, b, buf_v, sin)
                    else:
                        @pl.when(q < RPB // 2 - 1)
                        def _():
                            start_gather(r + 1, b, buf_v, sin)
                pltpu.async_copy(obuf_v, out_h.at[row], sout)
            return carry

        lax.fori_loop(0, RPB // 2, pair_body, 0)

        pltpu.make_async_copy(obuf0_v, out_h.at[rbase + RPB - 2], sout0).wait()
        pltpu.make_async_copy(obuf1_v, out_h.at[rbase + RPB - 1], sout1).wait()

    return sc_fn(ids, word_table, pos_table, tok_table, gamma, beta)
